# baseline (speedup 1.0000x reference)
import functools

import jax
import jax.numpy as jnp
from jax import lax
from jax.experimental import pallas as pl
from jax.experimental.pallas import tpu as pltpu
from jax.experimental.pallas import tpu_sc as plsc

B = 1024
L = 20
VOCAB = 100000
WE = 500
WEP = 512
PK = WEP // 2
RNN = 512
W2V = 500
BOW = 7807
OUT = RNN + W2V + BOW
BL = B * L


_VB = 2048


def _tr_body(tT_ref, out_ref):
    blk = tT_ref[:].T.astype(jnp.bfloat16)
    zpad = jnp.zeros((_VB, WEP - WE), jnp.bfloat16)
    full = jnp.concatenate([blk, zpad], axis=1)
    lo = lax.bitcast_convert_type(full[:, :PK], jnp.uint16).astype(jnp.uint32)
    hi = lax.bitcast_convert_type(full[:, PK:], jnp.uint16).astype(jnp.uint32)
    out_ref[:] = lax.bitcast_convert_type(lo | (hi << 16), jnp.int32)


_transpose = pl.pallas_call(
    _tr_body,
    grid=(pl.cdiv(VOCAB, _VB),),
    in_specs=[pl.BlockSpec((WE, _VB), lambda i: (0, i))],
    out_specs=pl.BlockSpec((_VB, PK), lambda i: (i, 0)),
    out_shape=jax.ShapeDtypeStruct((VOCAB, PK), jnp.int32),
)


_NC, _NS = 2, 16
_NW = _NC * _NS
_ROWS_PER_W = BL // _NW
_CHUNK = 128
_NCHUNK = _ROWS_PER_W // _CHUNK


def _gather_body(ids_hbm, table_hbm, out_hbm, idx_v, rows_v, sem):
    wid = lax.axis_index("s") * _NC + lax.axis_index("c")
    base = wid * _ROWS_PER_W
    for c in range(_NCHUNK):
        row0 = base + c * _CHUNK
        pltpu.sync_copy(ids_hbm.at[pl.ds(row0, _CHUNK)], idx_v)
        pltpu.async_copy(table_hbm.at[idx_v], rows_v, sem).wait()
        pltpu.sync_copy(rows_v, out_hbm.at[pl.ds(row0, _CHUNK)])


@functools.lru_cache(maxsize=1)
def _make_gather():
    return pl.kernel(
        _gather_body,
        out_type=jax.ShapeDtypeStruct((BL, PK), jnp.int32),
        mesh=plsc.VectorSubcoreMesh(core_axis_name="c", subcore_axis_name="s",
                                    num_cores=_NC, num_subcores=_NS),
        scratch_types=[
            pltpu.VMEM((_CHUNK,), jnp.int32),
            pltpu.VMEM((_CHUNK, PK), jnp.int32),
            pltpu.SemaphoreType.DMA,
        ],
    )


_BB = 256


def _unpack(px):
    u = lax.bitcast_convert_type(px, jnp.uint32)
    lo = lax.bitcast_convert_type((u & 0xFFFF).astype(jnp.uint16),
                                  jnp.bfloat16)
    hi = lax.bitcast_convert_type((u >> 16).astype(jnp.uint16),
                                  jnp.bfloat16)
    return lo, hi


def _concat_body(w2vT_ref, bow_ref, outT_ref):
    outT_ref[RNN:RNN + W2V, :] = w2vT_ref[:]
    outT_ref[RNN + W2V:, :] = bow_ref[:].T


_concat = pl.pallas_call(
    _concat_body,
    grid=(B // _BB,),
    in_specs=[
        pl.BlockSpec((W2V, _BB), lambda i: (0, i)),
        pl.BlockSpec((_BB, BOW), lambda i: (i, 0)),
    ],
    out_specs=pl.BlockSpec((OUT, _BB), lambda i: (0, i)),
    out_shape=jax.ShapeDtypeStruct((OUT, B), jnp.float32),
)


def _gru_body(x_ref, lenf_ref, wih_ref, whh_ref,
              bih_ref, bhh_ref, prev_ref, outT_ref):
    lenf = lenf_ref[:]
    bias = bih_ref[:] + bhh_ref[:]
    h = jnp.zeros((_BB, RNN), jnp.float32)
    acc = jnp.zeros((_BB, RNN), jnp.float32)
    for t in range(L):
        xlo, xhi = _unpack(x_ref[t])
        gi = (jnp.dot(xlo, wih_ref[:PK],
                      preferred_element_type=jnp.float32) +
              jnp.dot(xhi, wih_ref[PK:],
                      preferred_element_type=jnp.float32))
        gh = jnp.dot(h.astype(jnp.bfloat16), whh_ref[:],
                     preferred_element_type=jnp.float32)
        s = gi + gh + bias
        r = 0.5 * jnp.tanh(0.5 * s[:, :RNN]) + 0.5
        z = 0.5 * jnp.tanh(0.5 * s[:, RNN:2 * RNN]) + 0.5
        n = jnp.tanh(gi[:, 2 * RNN:] + bih_ref[:, 2 * RNN:] +
                     r * (gh[:, 2 * RNN:] + bhh_ref[:, 2 * RNN:]))
        h = (1.0 - z) * n + z * h
        acc = acc + jnp.where(lenf > t, h, 0.0)
    rnn_out = acc / lenf
    outT_ref[:] = rnn_out.T


_gru = pl.pallas_call(
    _gru_body,
    grid=(B // _BB,),
    in_specs=[
        pl.BlockSpec((L, _BB, PK), lambda i: (0, i, 0)),
        pl.BlockSpec((_BB, 1), lambda i: (i, 0)),
        pl.BlockSpec((WEP, 3 * RNN), lambda i: (0, 0)),
        pl.BlockSpec((RNN, 3 * RNN), lambda i: (0, 0)),
        pl.BlockSpec((1, 3 * RNN), lambda i: (0, 0)),
        pl.BlockSpec((1, 3 * RNN), lambda i: (0, 0)),
        pl.BlockSpec(memory_space=pl.ANY),
    ],
    out_specs=pl.BlockSpec((RNN, _BB), lambda i: (0, i)),
    out_shape=jax.ShapeDtypeStruct((OUT, B), jnp.float32),
    input_output_aliases={6: 0},
)


def kernel(token_ids, lengths, w2v_out, bow_out, emb_table, W_ih, W_hh,
           b_ih, b_hh):
    table_pk = _transpose(emb_table.T)
    ids_tm = token_ids.T.reshape(BL)
    x = _make_gather()(ids_tm, table_pk)
    x = x.reshape(L, B, PK)
    lenf = lengths.astype(jnp.float32).reshape(B, 1)
    wihT = jnp.pad(W_ih.T, ((0, WEP - WE), (0, 0))).astype(jnp.bfloat16)
    whhT = W_hh.T.astype(jnp.bfloat16)
    partial_out = _concat(w2v_out.T, bow_out)
    outT = _gru(x, lenf, wihT, whhT,
                b_ih.reshape(1, -1), b_hh.reshape(1, -1), partial_out)
    return outT.T


# speedup vs baseline: 1.0700x; 1.0700x over previous
import functools

import jax
import jax.numpy as jnp
from jax import lax
from jax.experimental import pallas as pl
from jax.experimental.pallas import tpu as pltpu
from jax.experimental.pallas import tpu_sc as plsc

B = 1024
L = 20
VOCAB = 100000
WE = 500
WEP = 512
PK = WEP // 2
RNN = 512
W2V = 500
BOW = 7807
OUT = RNN + W2V + BOW
BL = B * L


_VB = 4096


def _tr_body(tT_ref, out_ref):
    blk = tT_ref[:].T.astype(jnp.bfloat16)
    zpad = jnp.zeros((_VB, WEP - WE), jnp.bfloat16)
    full = jnp.concatenate([blk, zpad], axis=1)
    lo = lax.bitcast_convert_type(full[:, :PK], jnp.uint16).astype(jnp.uint32)
    hi = lax.bitcast_convert_type(full[:, PK:], jnp.uint16).astype(jnp.uint32)
    out_ref[:] = lax.bitcast_convert_type(lo | (hi << 16), jnp.int32)


_transpose = pl.pallas_call(
    _tr_body,
    grid=(pl.cdiv(VOCAB, _VB),),
    in_specs=[pl.BlockSpec((WE, _VB), lambda i: (0, i))],
    out_specs=pl.BlockSpec((_VB, PK), lambda i: (i, 0)),
    out_shape=jax.ShapeDtypeStruct((VOCAB, PK), jnp.int32),
)


_NC, _NS = 2, 16
_NW = _NC * _NS
_ROWS_PER_W = BL // _NW
_CHUNK = 128
_NCHUNK = _ROWS_PER_W // _CHUNK


def _gather_body(ids_hbm, table_hbm, out_hbm, idx_v, rows_v, sem):
    wid = lax.axis_index("s") * _NC + lax.axis_index("c")
    base = wid * _ROWS_PER_W
    for c in range(_NCHUNK):
        row0 = base + c * _CHUNK
        pltpu.sync_copy(ids_hbm.at[pl.ds(row0, _CHUNK)], idx_v)
        pltpu.async_copy(table_hbm.at[idx_v], rows_v, sem).wait()
        pltpu.sync_copy(rows_v, out_hbm.at[pl.ds(row0, _CHUNK)])


@functools.lru_cache(maxsize=1)
def _make_gather():
    return pl.kernel(
        _gather_body,
        out_type=jax.ShapeDtypeStruct((BL, PK), jnp.int32),
        mesh=plsc.VectorSubcoreMesh(core_axis_name="c", subcore_axis_name="s",
                                    num_cores=_NC, num_subcores=_NS),
        scratch_types=[
            pltpu.VMEM((_CHUNK,), jnp.int32),
            pltpu.VMEM((_CHUNK, PK), jnp.int32),
            pltpu.SemaphoreType.DMA,
        ],
    )


_BB = 256


def _unpack(px):
    u = lax.bitcast_convert_type(px, jnp.uint32)
    lo = lax.bitcast_convert_type((u & 0xFFFF).astype(jnp.uint16),
                                  jnp.bfloat16)
    hi = lax.bitcast_convert_type((u >> 16).astype(jnp.uint16),
                                  jnp.bfloat16)
    return lo, hi


def _gru_body(x_ref, lenf_ref, w2vT_ref, bow_ref, wih_ref, whh_ref,
              bih_ref, bhh_ref, outT_ref):
    lenf = lenf_ref[:]
    bias = bih_ref[:] + bhh_ref[:]
    h = jnp.zeros((_BB, RNN), jnp.float32)
    acc = jnp.zeros((_BB, RNN), jnp.float32)
    for t in range(L):
        xlo, xhi = _unpack(x_ref[t])
        gi = (jnp.dot(xlo, wih_ref[:PK],
                      preferred_element_type=jnp.float32) +
              jnp.dot(xhi, wih_ref[PK:],
                      preferred_element_type=jnp.float32))
        gh = jnp.dot(h.astype(jnp.bfloat16), whh_ref[:],
                     preferred_element_type=jnp.float32)
        s = gi + gh + bias
        r = 0.5 * jnp.tanh(0.5 * s[:, :RNN]) + 0.5
        z = 0.5 * jnp.tanh(0.5 * s[:, RNN:2 * RNN]) + 0.5
        n = jnp.tanh(gi[:, 2 * RNN:] + bih_ref[:, 2 * RNN:] +
                     r * (gh[:, 2 * RNN:] + bhh_ref[:, 2 * RNN:]))
        h = (1.0 - z) * n + z * h
        acc = acc + jnp.where(lenf > t, h, 0.0)
    rnn_out = acc / lenf
    outT_ref[:RNN, :] = rnn_out.T
    outT_ref[RNN:RNN + W2V, :] = w2vT_ref[:]
    outT_ref[RNN + W2V:, :] = bow_ref[:].T


_gru = pl.pallas_call(
    _gru_body,
    grid=(B // _BB,),
    in_specs=[
        pl.BlockSpec((L, _BB, PK), lambda i: (0, i, 0)),
        pl.BlockSpec((_BB, 1), lambda i: (i, 0)),
        pl.BlockSpec((W2V, _BB), lambda i: (0, i)),
        pl.BlockSpec((_BB, BOW), lambda i: (i, 0)),
        pl.BlockSpec((WEP, 3 * RNN), lambda i: (0, 0)),
        pl.BlockSpec((RNN, 3 * RNN), lambda i: (0, 0)),
        pl.BlockSpec((1, 3 * RNN), lambda i: (0, 0)),
        pl.BlockSpec((1, 3 * RNN), lambda i: (0, 0)),
    ],
    out_specs=pl.BlockSpec((OUT, _BB), lambda i: (0, i)),
    out_shape=jax.ShapeDtypeStruct((OUT, B), jnp.float32),
)


def kernel(token_ids, lengths, w2v_out, bow_out, emb_table, W_ih, W_hh,
           b_ih, b_hh):
    table_pk = _transpose(emb_table.T)
    ids_tm = token_ids.T.reshape(BL)
    x = _make_gather()(ids_tm, table_pk)
    x = x.reshape(L, B, PK)
    lenf = lengths.astype(jnp.float32).reshape(B, 1)
    wihT = jnp.pad(W_ih.T, ((0, WEP - WE), (0, 0))).astype(jnp.bfloat16)
    whhT = W_hh.T.astype(jnp.bfloat16)
    outT = _gru(x, lenf, w2v_out.T, bow_out,
                wihT, whhT, b_ih.reshape(1, -1), b_hh.reshape(1, -1))
    return outT.T


# speedup vs baseline: 1.0837x; 1.0128x over previous
import functools

import jax
import jax.numpy as jnp
from jax import lax
from jax.experimental import pallas as pl
from jax.experimental.pallas import tpu as pltpu
from jax.experimental.pallas import tpu_sc as plsc

B = 1024
L = 20
VOCAB = 100000
WE = 500
WEP = 512
PK = WEP // 2
RNN = 512
W2V = 500
BOW = 7807
OUT = RNN + W2V + BOW
BL = B * L


_VB = 8192


def _tr_body(tT_ref, out_ref):
    blk = tT_ref[:].T.astype(jnp.bfloat16)
    zpad = jnp.zeros((_VB, WEP - WE), jnp.bfloat16)
    full = jnp.concatenate([blk, zpad], axis=1)
    lo = lax.bitcast_convert_type(full[:, :PK], jnp.uint16).astype(jnp.uint32)
    hi = lax.bitcast_convert_type(full[:, PK:], jnp.uint16).astype(jnp.uint32)
    out_ref[:] = lax.bitcast_convert_type(lo | (hi << 16), jnp.int32)


_transpose = pl.pallas_call(
    _tr_body,
    grid=(pl.cdiv(VOCAB, _VB),),
    in_specs=[pl.BlockSpec((WE, _VB), lambda i: (0, i))],
    out_specs=pl.BlockSpec((_VB, PK), lambda i: (i, 0)),
    out_shape=jax.ShapeDtypeStruct((VOCAB, PK), jnp.int32),
)


_NC, _NS = 2, 16
_NW = _NC * _NS
_ROWS_PER_W = BL // _NW
_CHUNK = 128
_NCHUNK = _ROWS_PER_W // _CHUNK


def _gather_body(ids_hbm, table_hbm, out_hbm, idx_v, rows_v, sem):
    wid = lax.axis_index("s") * _NC + lax.axis_index("c")
    base = wid * _ROWS_PER_W
    for c in range(_NCHUNK):
        row0 = base + c * _CHUNK
        pltpu.sync_copy(ids_hbm.at[pl.ds(row0, _CHUNK)], idx_v)
        pltpu.async_copy(table_hbm.at[idx_v], rows_v, sem).wait()
        pltpu.sync_copy(rows_v, out_hbm.at[pl.ds(row0, _CHUNK)])


@functools.lru_cache(maxsize=1)
def _make_gather():
    return pl.kernel(
        _gather_body,
        out_type=jax.ShapeDtypeStruct((BL, PK), jnp.int32),
        mesh=plsc.VectorSubcoreMesh(core_axis_name="c", subcore_axis_name="s",
                                    num_cores=_NC, num_subcores=_NS),
        scratch_types=[
            pltpu.VMEM((_CHUNK,), jnp.int32),
            pltpu.VMEM((_CHUNK, PK), jnp.int32),
            pltpu.SemaphoreType.DMA,
        ],
    )


_BB = 256


def _unpack(px):
    u = lax.bitcast_convert_type(px, jnp.uint32)
    lo = lax.bitcast_convert_type((u & 0xFFFF).astype(jnp.uint16),
                                  jnp.bfloat16)
    hi = lax.bitcast_convert_type((u >> 16).astype(jnp.uint16),
                                  jnp.bfloat16)
    return lo, hi


def _gru_body(x_ref, lenf_ref, w2vT_ref, bow_ref, wih_ref, whh_ref,
              bih_ref, bhh_ref, outT_ref):
    lenf = lenf_ref[:]
    bias = bih_ref[:] + bhh_ref[:]
    h = jnp.zeros((_BB, RNN), jnp.float32)
    acc = jnp.zeros((_BB, RNN), jnp.float32)
    for t in range(L):
        xlo, xhi = _unpack(x_ref[t])
        gi = (jnp.dot(xlo, wih_ref[:PK],
                      preferred_element_type=jnp.float32) +
              jnp.dot(xhi, wih_ref[PK:],
                      preferred_element_type=jnp.float32))
        gh = jnp.dot(h.astype(jnp.bfloat16), whh_ref[:],
                     preferred_element_type=jnp.float32)
        s = gi + gh + bias
        r = 0.5 * jnp.tanh(0.5 * s[:, :RNN]) + 0.5
        z = 0.5 * jnp.tanh(0.5 * s[:, RNN:2 * RNN]) + 0.5
        n = jnp.tanh(gi[:, 2 * RNN:] + bih_ref[:, 2 * RNN:] +
                     r * (gh[:, 2 * RNN:] + bhh_ref[:, 2 * RNN:]))
        h = (1.0 - z) * n + z * h
        acc = acc + jnp.where(lenf > t, h, 0.0)
    rnn_out = acc / lenf
    outT_ref[:RNN, :] = rnn_out.T
    outT_ref[RNN:RNN + W2V, :] = w2vT_ref[:]
    outT_ref[RNN + W2V:, :] = bow_ref[:].T


_gru = pl.pallas_call(
    _gru_body,
    grid=(B // _BB,),
    in_specs=[
        pl.BlockSpec((L, _BB, PK), lambda i: (0, i, 0)),
        pl.BlockSpec((_BB, 1), lambda i: (i, 0)),
        pl.BlockSpec((W2V, _BB), lambda i: (0, i)),
        pl.BlockSpec((_BB, BOW), lambda i: (i, 0)),
        pl.BlockSpec((WEP, 3 * RNN), lambda i: (0, 0)),
        pl.BlockSpec((RNN, 3 * RNN), lambda i: (0, 0)),
        pl.BlockSpec((1, 3 * RNN), lambda i: (0, 0)),
        pl.BlockSpec((1, 3 * RNN), lambda i: (0, 0)),
    ],
    out_specs=pl.BlockSpec((OUT, _BB), lambda i: (0, i)),
    out_shape=jax.ShapeDtypeStruct((OUT, B), jnp.float32),
)


def kernel(token_ids, lengths, w2v_out, bow_out, emb_table, W_ih, W_hh,
           b_ih, b_hh):
    table_pk = _transpose(emb_table.T)
    ids_tm = token_ids.T.reshape(BL)
    x = _make_gather()(ids_tm, table_pk)
    x = x.reshape(L, B, PK)
    lenf = lengths.astype(jnp.float32).reshape(B, 1)
    wihT = jnp.pad(W_ih.T, ((0, WEP - WE), (0, 0))).astype(jnp.bfloat16)
    whhT = W_hh.T.astype(jnp.bfloat16)
    outT = _gru(x, lenf, w2v_out.T, bow_out,
                wihT, whhT, b_ih.reshape(1, -1), b_hh.reshape(1, -1))
    return outT.T
